# baseline (speedup 1.0000x reference)
import functools

import jax
import jax.numpy as jnp
from jax import lax
from jax.experimental import pallas as pl
from jax.experimental.pallas import tpu as pltpu
from jax.experimental.pallas import tpu_sc as plsc

CAP = 1_000_000
D = 64
NC, NS = 2, 16
NW = NC * NS
CHUNK = 496
NCHUNK = 35
RPT = CHUNK * NCHUNK
SC_ROWS = NW * RPT
TC_ROWS = CAP - SC_ROWS
TC_BLK = 7408
TC_GRID = TC_ROWS // TC_BLK
NEG_INF = float("-inf")


def _k1sc_body(x_hbm, mem_hbm, part_hbm, xv, buf0, buf1, pv, sem0, sem1):
    c = lax.axis_index("c")
    s = lax.axis_index("s")
    wid = s * NC + c
    base = wid * RPT

    pltpu.sync_copy(x_hbm, xv)
    x0 = xv[pl.ds(0, 16)]
    x1 = xv[pl.ds(16, 16)]
    x2 = xv[pl.ds(32, 16)]
    x3 = xv[pl.ds(48, 16)]

    bufs = (buf0, buf1)
    sems = (sem0, sem1)

    def start(k, parity):
        off = pl.multiple_of(base + k * CHUNK, 8)
        return pltpu.async_copy(
            mem_hbm.at[pl.ds(off, CHUNK)], bufs[parity], sems[parity])

    def row_body(buf, chunk_base, r, carry):
        bd2, bs, bi, bd = carry
        m0 = buf[r, pl.ds(0, 16)]
        m1 = buf[r, pl.ds(16, 16)]
        m2 = buf[r, pl.ds(32, 16)]
        m3 = buf[r, pl.ds(48, 16)]
        dv = m0 * x0 + m1 * x1 + m2 * x2 + m3 * x3
        sv = m0 * m0 + m1 * m1 + m2 * m2 + m3 * m3
        d = jnp.sum(dv)
        sq = jnp.maximum(jnp.sum(sv), jnp.float32(1e-30))
        d2 = d * jnp.abs(d)
        gi = chunk_base + r
        lhs = d2 * bs
        rhs = bd2 * sq
        better = (lhs > rhs) | ((lhs == rhs) & (gi < bi))
        return (jnp.where(better, d2, bd2),
                jnp.where(better, sq, bs),
                jnp.where(better, gi, bi),
                jnp.where(better, d, bd))

    carry = (jnp.float32(NEG_INF), jnp.float32(1.0), jnp.int32(0), jnp.float32(0.0))

    def chunk_sweep(buf, chunk_base, carry):
        body = functools.partial(row_body, buf, chunk_base)
        return lax.fori_loop(0, CHUNK, body, carry, unroll=4)

    handles = [start(0, 0), start(1, 1)]
    for k in range(NCHUNK):
        handles[k % 2].wait()
        carry = chunk_sweep(bufs[k % 2], base + k * CHUNK, carry)
        if k + 2 < NCHUNK:
            handles[k % 2] = start(k + 2, k % 2)

    bd2, bs, bi, bd = carry
    lanes = lax.iota(jnp.int32, 16)
    out = jnp.where(lanes == 0, bd2,
          jnp.where(lanes == 1, bi.astype(jnp.float32),
          jnp.where(lanes == 2, bd,
          jnp.where(lanes == 3, bs, jnp.float32(0.0)))))
    dummy = jnp.where(lanes == 0, NEG_INF,
            jnp.where(lanes == 1, jnp.float32(2.0e9),
            jnp.where(lanes == 3, jnp.float32(1.0), jnp.float32(0.0))))
    pv[0, :] = out
    for j in range(1, 8):
        pv[j, :] = dummy
    pltpu.sync_copy(pv, part_hbm.at[pl.ds(wid * 8, 8)])


@functools.cache
def _get_k1sc():
    return pl.kernel(
        _k1sc_body,
        out_type=jax.ShapeDtypeStruct((NW * 8, 16), jnp.float32),
        mesh=plsc.VectorSubcoreMesh(
            core_axis_name="c", subcore_axis_name="s",
            num_cores=NC, num_subcores=NS),
        scratch_types=[
            pltpu.VMEM((D,), jnp.float32),
            pltpu.VMEM((CHUNK, D), jnp.float32),
            pltpu.VMEM((CHUNK, D), jnp.float32),
            pltpu.VMEM((8, 16), jnp.float32),
            pltpu.SemaphoreType.DMA,
            pltpu.SemaphoreType.DMA,
        ],
        compiler_params=pltpu.CompilerParams(needs_layout_passes=False),
    )


def _k1tc_body(x_ref, m_ref, o_ref):
    i = pl.program_id(0)
    m = m_ref[...]
    xcol = x_ref[...]
    ones = jnp.ones((D, 1), jnp.float32)
    dots = jax.lax.dot_general(
        m, xcol, (((1,), (0,)), ((), ())),
        preferred_element_type=jnp.float32)
    sq = jax.lax.dot_general(
        m * m, ones, (((1,), (0,)), ((), ())),
        preferred_element_type=jnp.float32)
    sq = jnp.maximum(sq, jnp.float32(1e-30))
    f = dots * jnp.abs(dots) / sq
    fm = jnp.max(f)
    iot = lax.broadcasted_iota(jnp.int32, (TC_BLK, 1), 0)
    li = jnp.min(jnp.where(f == fm, iot, jnp.int32(2**31 - 1)))
    gi = (SC_ROWS + i * TC_BLK + li).astype(jnp.float32)
    ld = jnp.sum(jnp.where(iot == li, dots, 0.0))
    ls = jnp.sum(jnp.where(iot == li, sq, 0.0))
    lanes = lax.broadcasted_iota(jnp.int32, (1, 1, 16), 2)
    o_ref[...] = jnp.where(lanes == 0, fm * ls,
                 jnp.where(lanes == 1, gi,
                 jnp.where(lanes == 2, ld,
                 jnp.where(lanes == 3, ls, jnp.float32(0.0)))))


_k1tc = pl.pallas_call(
    _k1tc_body,
    grid=(TC_GRID,),
    in_specs=[
        pl.BlockSpec((D, 1), lambda i: (0, 0)),
        pl.BlockSpec((TC_BLK, D), lambda i: (i + SC_ROWS // TC_BLK, 0)),
    ],
    out_specs=pl.BlockSpec((1, 1, 16), lambda i: (i, 0, 0)),
    out_shape=jax.ShapeDtypeStruct((TC_GRID, 1, 16), jnp.float32),
)


def _k2_body(part_ref, x_ref, out_ref):
    p = part_ref[...]
    x = x_ref[...]
    xn = jnp.sqrt(jnp.sum(x * x))
    d2 = p[:, 0]
    idxf = p[:, 1]
    d = p[:, 2]
    sq = p[:, 3]
    f = d2 / sq
    fmax = jnp.max(f)
    ismax = f == fmax
    gidx_f = jnp.min(jnp.where(ismax, idxf, jnp.float32(2**31)))
    sel = ismax & (idxf == gidx_f)
    dw = jnp.sum(jnp.where(sel, d, 0.0))
    sw = jnp.sum(jnp.where(sel, sq, 0.0))
    val = dw / jnp.maximum(jnp.sqrt(sw) * xn, jnp.float32(1e-8))
    gidx = gidx_f.astype(jnp.int32)
    rows = lax.broadcasted_iota(jnp.int32, (CAP // D, D), 0)
    cols = lax.broadcasted_iota(jnp.int32, (CAP // D, D), 1)
    hit = (rows == lax.shift_right_logical(gidx, 6)) & (cols == (gidx & 63))
    out_ref[...] = jnp.where(hit, val, jnp.float32(0.0))


_k2 = pl.pallas_call(
    _k2_body,
    out_shape=jax.ShapeDtypeStruct((CAP // D, D), jnp.float32),
)


def kernel(x, memory):
    x2d = x.reshape(1, D)
    part_sc = _get_k1sc()(x, memory)
    part_tc = _k1tc(x.reshape(D, 1), memory).reshape(TC_GRID, 16)
    parts = jnp.concatenate([part_sc, part_tc], axis=0)
    out2d = _k2(parts, x2d)
    return out2d.reshape(-1)


# speedup vs baseline: 1.1068x; 1.1068x over previous
import functools

import jax
import jax.numpy as jnp
from jax import lax
from jax.experimental import pallas as pl
from jax.experimental.pallas import tpu as pltpu
from jax.experimental.pallas import tpu_sc as plsc

CAP = 1_000_000
D = 64
NC, NS = 2, 16
NW = NC * NS
CHUNK = 496
NCHUNK = 63
RPT = CHUNK * NCHUNK
SC_ROWS = NW * RPT
TAIL = CAP - SC_ROWS
NEG_INF = float("-inf")


def _k1sc_body(x_hbm, mem_hbm, part_hbm, xv, buf0, buf1, pv, sem0, sem1):
    c = lax.axis_index("c")
    s = lax.axis_index("s")
    wid = s * NC + c
    base = wid * RPT

    pltpu.sync_copy(x_hbm, xv)
    x0 = xv[pl.ds(0, 16)]
    x1 = xv[pl.ds(16, 16)]
    x2 = xv[pl.ds(32, 16)]
    x3 = xv[pl.ds(48, 16)]

    bufs = (buf0, buf1)
    sems = (sem0, sem1)

    def start(k, parity):
        off = pl.multiple_of(base + k * CHUNK, 8)
        return pltpu.async_copy(
            mem_hbm.at[pl.ds(off, CHUNK)], bufs[parity], sems[parity])

    def row_body(buf, chunk_base, r, carry):
        bd2, bs, bi, bd = carry
        m0 = buf[r, pl.ds(0, 16)]
        m1 = buf[r, pl.ds(16, 16)]
        m2 = buf[r, pl.ds(32, 16)]
        m3 = buf[r, pl.ds(48, 16)]
        dv = m0 * x0 + m1 * x1 + m2 * x2 + m3 * x3
        sv = m0 * m0 + m1 * m1 + m2 * m2 + m3 * m3
        d = jnp.sum(dv)
        sq = jnp.maximum(jnp.sum(sv), jnp.float32(1e-30))
        d2 = d * jnp.abs(d)
        gi = chunk_base + r
        lhs = d2 * bs
        rhs = bd2 * sq
        better = (lhs > rhs) | ((lhs == rhs) & (gi < bi))
        return (jnp.where(better, d2, bd2),
                jnp.where(better, sq, bs),
                jnp.where(better, gi, bi),
                jnp.where(better, d, bd))

    carry = (jnp.float32(NEG_INF), jnp.float32(1.0), jnp.int32(0), jnp.float32(0.0))

    def chunk_sweep(buf, chunk_base, carry):
        body = functools.partial(row_body, buf, chunk_base)
        return lax.fori_loop(0, CHUNK, body, carry, unroll=4)

    def wait(parity):
        pltpu.make_async_copy(
            mem_hbm.at[pl.ds(0, CHUNK)], bufs[parity], sems[parity]).wait()

    start(0, 0)
    start(1, 1)

    def outer(j, cy):
        k0 = j * 2
        wait(0)
        cy = chunk_sweep(buf0, base + k0 * CHUNK, cy)

        @pl.when(k0 + 2 < NCHUNK)
        def _():
            start(k0 + 2, 0)

        wait(1)
        cy = chunk_sweep(buf1, base + (k0 + 1) * CHUNK, cy)

        @pl.when(k0 + 3 < NCHUNK)
        def _():
            start(k0 + 3, 1)

        return cy

    carry = lax.fori_loop(0, NCHUNK // 2, outer, carry)
    wait(0)
    carry = chunk_sweep(buf0, base + (NCHUNK - 1) * CHUNK, carry)

    pltpu.sync_copy(mem_hbm.at[pl.ds(SC_ROWS, TAIL)], buf1.at[pl.ds(0, TAIL)])

    def tail_step(r, cy):
        return row_body(buf1, SC_ROWS, r, cy)

    carry = lax.cond(wid == NW - 1,
                     lambda cy: lax.fori_loop(0, TAIL, tail_step, cy),
                     lambda cy: cy, carry)

    bd2, bs, bi, bd = carry
    lanes = lax.iota(jnp.int32, 16)
    out = jnp.where(lanes == 0, bd2,
          jnp.where(lanes == 1, bi.astype(jnp.float32),
          jnp.where(lanes == 2, bd,
          jnp.where(lanes == 3, bs, jnp.float32(0.0)))))
    dummy = jnp.where(lanes == 0, NEG_INF,
            jnp.where(lanes == 1, jnp.float32(2.0e9),
            jnp.where(lanes == 3, jnp.float32(1.0), jnp.float32(0.0))))
    pv[0, :] = out
    for j in range(1, 8):
        pv[j, :] = dummy
    pltpu.sync_copy(pv, part_hbm.at[pl.ds(wid * 8, 8)])


@functools.cache
def _get_k1sc():
    return pl.kernel(
        _k1sc_body,
        out_type=jax.ShapeDtypeStruct((NW * 8, 16), jnp.float32),
        mesh=plsc.VectorSubcoreMesh(
            core_axis_name="c", subcore_axis_name="s",
            num_cores=NC, num_subcores=NS),
        scratch_types=[
            pltpu.VMEM((D,), jnp.float32),
            pltpu.VMEM((CHUNK, D), jnp.float32),
            pltpu.VMEM((CHUNK, D), jnp.float32),
            pltpu.VMEM((8, 16), jnp.float32),
            pltpu.SemaphoreType.DMA,
            pltpu.SemaphoreType.DMA,
        ],
        compiler_params=pltpu.CompilerParams(needs_layout_passes=False),
    )


def _k2_body(part_ref, x_ref, out_ref):
    p = part_ref[...]
    x = x_ref[...]
    xn = jnp.sqrt(jnp.sum(x * x))
    d2 = p[:, 0]
    idxf = p[:, 1]
    d = p[:, 2]
    sq = p[:, 3]
    f = d2 / sq
    fmax = jnp.max(f)
    ismax = f == fmax
    gidx_f = jnp.min(jnp.where(ismax, idxf, jnp.float32(2**31)))
    sel = ismax & (idxf == gidx_f)
    dw = jnp.sum(jnp.where(sel, d, 0.0))
    sw = jnp.sum(jnp.where(sel, sq, 0.0))
    val = dw / jnp.maximum(jnp.sqrt(sw) * xn, jnp.float32(1e-8))
    gidx = gidx_f.astype(jnp.int32)
    rows = lax.broadcasted_iota(jnp.int32, (CAP // D, D), 0)
    cols = lax.broadcasted_iota(jnp.int32, (CAP // D, D), 1)
    hit = (rows == lax.shift_right_logical(gidx, 6)) & (cols == (gidx & 63))
    out_ref[...] = jnp.where(hit, val, jnp.float32(0.0))


_k2 = pl.pallas_call(
    _k2_body,
    out_shape=jax.ShapeDtypeStruct((CAP // D, D), jnp.float32),
)


def kernel(x, memory):
    x2d = x.reshape(1, D)
    parts = _get_k1sc()(x, memory)
    out2d = _k2(parts, x2d)
    return out2d.reshape(-1)
